# Initial kernel scaffold; baseline (speedup 1.0000x reference)
import jax
import jax.numpy as jnp
from jax.experimental import pallas as pl


def kernel(x_prev, c_prev, A, Pid, s_prev):
    raise NotImplementedError("write your pallas kernel here")


# speedup vs baseline: 60.4804x; 60.4804x over previous
import jax
import jax.numpy as jnp
from jax.experimental import pallas as pl
from jax.experimental.pallas import tpu as pltpu

V = 512
E = 4096
F = V + E
VB = 512
NB = F // VB
NEB = E // VB
FX = 128
FC = 64


def _meta_body(a_ref, pid_ref, u_ref, rank_ref, meta_ref):
    a = a_ref[0]
    row = jax.lax.broadcasted_iota(jnp.int32, (V, V), 0)
    col = jax.lax.broadcasted_iota(jnp.int32, (V, V), 1)
    u = jnp.where((a != 0) & (col > row), 1.0, 0.0).astype(jnp.float32)
    ut = jnp.where(row <= col, 1.0, 0.0).astype(jnp.float32)
    rank = jnp.floor(jnp.dot(u, ut, preferred_element_type=jnp.float32) + 0.5)
    cnt = jnp.sum(u, axis=1)
    sl = jnp.where(row < col, 1.0, 0.0).astype(jnp.float32)
    off = jnp.floor(jnp.sum(cnt[:, None] * sl, axis=0) + 0.5)
    poly = jnp.max(pid_ref[0].astype(jnp.float32), axis=0)
    u_ref[0] = u
    rank_ref[0] = rank
    meta = jnp.concatenate(
        [cnt[None], off[None], poly[None], jnp.zeros((5, V), jnp.float32)], axis=0)
    meta_ref[0] = meta


def _one_i(meta_ref, eb):
    cnt = meta_ref[0, 0, :]
    off = meta_ref[0, 1, :]
    poly = meta_ref[0, 2, :]
    e_row = (jax.lax.broadcasted_iota(jnp.int32, (VB, V), 0)
             + eb * VB).astype(jnp.float32)
    one_i = jnp.where(
        (e_row > off[None, :] - 0.5) & (e_row < (off + cnt)[None, :] - 0.5),
        1.0, 0.0)
    pm = jnp.sum(one_i * poly[None, :], axis=1, keepdims=True)
    return one_i, pm, e_row, off


def _build_m(u_ref, rank_ref, meta_ref, eb):
    one_i, pm, e_row, off = _one_i(meta_ref, eb)
    rank_rows = jnp.floor(
        jnp.dot(one_i, rank_ref[0], preferred_element_type=jnp.float32) + 0.5)
    u_rows = jnp.dot(one_i, u_ref[0], preferred_element_type=jnp.float32)
    oe = jnp.sum(one_i * off[None, :], axis=1, keepdims=True)
    tgt = e_row[:, 0:1] - oe + 1.0
    one_j = jnp.where((u_rows > 0.5) & (jnp.abs(rank_rows - tgt) < 0.5),
                      1.0, 0.0)
    return one_i + one_j, pm


def _main_body(u_ref, rank_ref, meta_ref, x_ref, cf_ref, s_ref,
               x_out, c_out, s_out, a_out, p_out, m_st):
    bi = pl.program_id(0)
    bj = pl.program_id(1)

    @pl.when((bi == 0) & (bj == 0))
    def _corner():
        z = jnp.zeros((VB, VB), jnp.float32)
        a_out[0] = z
        p_out[0] = z
        x_out[0] = x_ref[0]
        c_out[0] = cf_ref[0]
        s_out[0] = s_ref[0]

    @pl.when((bi == 0) & (bj > 0))
    def _top_band():
        eb = bj - 1
        m, pm = _build_m(u_ref, rank_ref, meta_ref, eb)
        eye = jnp.where(
            jax.lax.broadcasted_iota(jnp.int32, (VB, VB), 0)
            == jax.lax.broadcasted_iota(jnp.int32, (VB, VB), 1), 1.0, 0.0)
        tn = (((0,), (0,)), ((), ()))
        a_out[0] = jax.lax.dot_general(m, eye, tn,
                                       preferred_element_type=jnp.float32)
        p_out[0] = jax.lax.dot_general(m * pm, eye, tn,
                                       preferred_element_type=jnp.float32)
        m_st[pl.ds(eb, 1)] = m[None]

    @pl.when((bi > 0) & (bj == 0))
    def _left_band():
        eb = bi - 1
        m = m_st[pl.ds(eb, 1)][0]
        _, pm, _, _ = _one_i(meta_ref, eb)
        a_out[0] = m
        p_out[0] = m * pm
        x_out[0] = 0.5 * jnp.dot(m, x_ref[0], preferred_element_type=jnp.float32)
        c_out[0] = 0.5 * jnp.dot(m, cf_ref[0], preferred_element_type=jnp.float32)
        s_out[0] = 0.5 * jnp.dot(m, s_ref[0], preferred_element_type=jnp.float32)

    @pl.when((bi > 0) & (bj > 0))
    def _bulk_zero():
        z = jnp.zeros((VB, VB), jnp.float32)
        a_out[0] = z
        p_out[0] = z


def kernel(x_prev, c_prev, A, Pid, s_prev):
    f32 = jnp.float32
    u, rank, meta = pl.pallas_call(
        _meta_body,
        out_shape=(
            jax.ShapeDtypeStruct((1, V, V), f32),
            jax.ShapeDtypeStruct((1, V, V), f32),
            jax.ShapeDtypeStruct((1, 8, V), f32),
        ),
    )(A, Pid)

    const = lambda i, j: (0, 0, 0)
    rowblk = lambda i, j: (0, i, 0)
    outs = pl.pallas_call(
        _main_body,
        grid=(NB, NB),
        in_specs=[
            pl.BlockSpec((1, V, V), const),
            pl.BlockSpec((1, V, V), const),
            pl.BlockSpec((1, 8, V), const),
            pl.BlockSpec((1, V, FX), const),
            pl.BlockSpec((1, V, FC), const),
            pl.BlockSpec((1, V, FX), const),
        ],
        out_specs=[
            pl.BlockSpec((1, VB, FX), rowblk),
            pl.BlockSpec((1, VB, FC), rowblk),
            pl.BlockSpec((1, VB, FX), rowblk),
            pl.BlockSpec((1, VB, VB), lambda i, j: (0, i, j)),
            pl.BlockSpec((1, VB, VB), lambda i, j: (0, i, j)),
        ],
        out_shape=[
            jax.ShapeDtypeStruct((1, F, FX), f32),
            jax.ShapeDtypeStruct((1, F, FC), f32),
            jax.ShapeDtypeStruct((1, F, FX), f32),
            jax.ShapeDtypeStruct((1, F, F), f32),
            jax.ShapeDtypeStruct((1, F, F), f32),
        ],
        scratch_shapes=[pltpu.VMEM((NEB, VB, VB), f32)],
    )(u, rank, meta, x_prev, c_prev, s_prev)
    x_new, c_new, s_new, a_new, p_new = outs
    return (x_new, c_new, a_new, p_new, s_new)


# speedup vs baseline: 62.6890x; 1.0365x over previous
import jax
import jax.numpy as jnp
from jax.experimental import pallas as pl
from jax.experimental.pallas import tpu as pltpu

V = 512
E = 4096
F = V + E
VB = 512
NB = F // VB
NEB = E // VB
FX = 128
FC = 64


def _meta_body(a_ref, pid_ref, u_ref, rank_ref, meta_ref):
    a = a_ref[0]
    row = jax.lax.broadcasted_iota(jnp.int32, (V, V), 0)
    col = jax.lax.broadcasted_iota(jnp.int32, (V, V), 1)
    u = jnp.where((a != 0) & (col > row), 1.0, 0.0).astype(jnp.float32)
    ut = jnp.where(row <= col, 1.0, 0.0).astype(jnp.float32)
    rank = jnp.floor(jnp.dot(u, ut, preferred_element_type=jnp.float32) + 0.5)
    cnt = jnp.sum(u, axis=1)
    sl = jnp.where(row < col, 1.0, 0.0).astype(jnp.float32)
    off = jnp.floor(jnp.sum(cnt[:, None] * sl, axis=0) + 0.5)
    poly = jnp.max(pid_ref[0].astype(jnp.float32), axis=0)
    u_ref[0] = u
    rank_ref[0] = rank
    meta = jnp.concatenate(
        [cnt[None], off[None], poly[None], jnp.zeros((5, V), jnp.float32)], axis=0)
    meta_ref[0] = meta


def _one_i(meta_ref, eb):
    cnt = meta_ref[0, 0, :]
    off = meta_ref[0, 1, :]
    poly = meta_ref[0, 2, :]
    e_row = (jax.lax.broadcasted_iota(jnp.int32, (VB, V), 0)
             + eb * VB).astype(jnp.float32)
    one_i = jnp.where(
        (e_row > off[None, :] - 0.5) & (e_row < (off + cnt)[None, :] - 0.5),
        1.0, 0.0)
    pm = jnp.sum(one_i * poly[None, :], axis=1, keepdims=True)
    return one_i, pm, e_row, off


def _build_m(u_ref, rank_ref, meta_ref, eb):
    one_i, pm, e_row, off = _one_i(meta_ref, eb)
    rank_rows = jnp.floor(
        jnp.dot(one_i, rank_ref[0], preferred_element_type=jnp.float32) + 0.5)
    u_rows = rank_rows - jnp.concatenate(
        [jnp.zeros((VB, 1), jnp.float32), rank_rows[:, : V - 1]], axis=1)
    oe = jnp.sum(one_i * off[None, :], axis=1, keepdims=True)
    tgt = e_row[:, 0:1] - oe + 1.0
    one_j = jnp.where((u_rows > 0.5) & (jnp.abs(rank_rows - tgt) < 0.5),
                      1.0, 0.0)
    return one_i + one_j, pm


def _main_body(u_ref, rank_ref, meta_ref, x_ref, cf_ref, s_ref,
               x_out, c_out, s_out, a_out, p_out, m_st):
    bi = pl.program_id(0)
    bj = pl.program_id(1)

    @pl.when((bi == 0) & (bj == 0))
    def _corner():
        z = jnp.zeros((VB, VB), jnp.float32)
        a_out[0] = z
        p_out[0] = z
        x_out[0] = x_ref[0]
        c_out[0] = cf_ref[0]
        s_out[0] = s_ref[0]

    @pl.when((bi == 0) & (bj > 0))
    def _top_band():
        eb = bj - 1
        m, pm = _build_m(u_ref, rank_ref, meta_ref, eb)
        mt = jnp.transpose(m)
        a_out[0] = mt
        p_out[0] = mt * jnp.transpose(pm)
        m_st[pl.ds(eb, 1)] = m[None]

    @pl.when((bi > 0) & (bj == 0))
    def _left_band():
        eb = bi - 1
        m = m_st[pl.ds(eb, 1)][0]
        _, pm, _, _ = _one_i(meta_ref, eb)
        a_out[0] = m
        p_out[0] = m * pm
        x_out[0] = 0.5 * jnp.dot(m, x_ref[0], preferred_element_type=jnp.float32)
        c_out[0] = 0.5 * jnp.dot(m, cf_ref[0], preferred_element_type=jnp.float32)
        s_out[0] = 0.5 * jnp.dot(m, s_ref[0], preferred_element_type=jnp.float32)

    @pl.when((bi > 0) & (bj > 0))
    def _bulk_zero():
        z = jnp.zeros((VB, VB), jnp.float32)
        a_out[0] = z
        p_out[0] = z


def kernel(x_prev, c_prev, A, Pid, s_prev):
    f32 = jnp.float32
    u, rank, meta = pl.pallas_call(
        _meta_body,
        out_shape=(
            jax.ShapeDtypeStruct((1, V, V), f32),
            jax.ShapeDtypeStruct((1, V, V), f32),
            jax.ShapeDtypeStruct((1, 8, V), f32),
        ),
    )(A, Pid)

    const = lambda i, j: (0, 0, 0)
    rowblk = lambda i, j: (0, i, 0)
    outs = pl.pallas_call(
        _main_body,
        grid=(NB, NB),
        in_specs=[
            pl.BlockSpec((1, V, V), const),
            pl.BlockSpec((1, V, V), const),
            pl.BlockSpec((1, 8, V), const),
            pl.BlockSpec((1, V, FX), const),
            pl.BlockSpec((1, V, FC), const),
            pl.BlockSpec((1, V, FX), const),
        ],
        out_specs=[
            pl.BlockSpec((1, VB, FX), rowblk),
            pl.BlockSpec((1, VB, FC), rowblk),
            pl.BlockSpec((1, VB, FX), rowblk),
            pl.BlockSpec((1, VB, VB), lambda i, j: (0, i, j)),
            pl.BlockSpec((1, VB, VB), lambda i, j: (0, i, j)),
        ],
        out_shape=[
            jax.ShapeDtypeStruct((1, F, FX), f32),
            jax.ShapeDtypeStruct((1, F, FC), f32),
            jax.ShapeDtypeStruct((1, F, FX), f32),
            jax.ShapeDtypeStruct((1, F, F), f32),
            jax.ShapeDtypeStruct((1, F, F), f32),
        ],
        scratch_shapes=[pltpu.VMEM((NEB, VB, VB), f32)],
    )(u, rank, meta, x_prev, c_prev, s_prev)
    x_new, c_new, s_new, a_new, p_new = outs
    return (x_new, c_new, a_new, p_new, s_new)


# speedup vs baseline: 64.8272x; 1.0341x over previous
import jax
import jax.numpy as jnp
from jax.experimental import pallas as pl
from jax.experimental.pallas import tpu as pltpu

V = 512
E = 4096
F = V + E
VB = 512
NB = F // VB
NEB = E // VB
FX = 128
FC = 64


def _one_i(meta_ref, eb):
    cnt = meta_ref[0, :]
    off = meta_ref[1, :]
    poly = meta_ref[2, :]
    e_row = (jax.lax.broadcasted_iota(jnp.int32, (VB, V), 0)
             + eb * VB).astype(jnp.float32)
    one_i = jnp.where(
        (e_row > off[None, :] - 0.5) & (e_row < (off + cnt)[None, :] - 0.5),
        1.0, 0.0)
    pm = jnp.sum(one_i * poly[None, :], axis=1, keepdims=True)
    return one_i, pm, e_row, off


def _build_m(u_ref, rank_ref, meta_ref, eb):
    one_i, pm, e_row, off = _one_i(meta_ref, eb)
    rank_rows = jnp.floor(
        jnp.dot(one_i, rank_ref[...], preferred_element_type=jnp.float32) + 0.5)
    u_rows = rank_rows - jnp.concatenate(
        [jnp.zeros((VB, 1), jnp.float32), rank_rows[:, : V - 1]], axis=1)
    oe = jnp.sum(one_i * off[None, :], axis=1, keepdims=True)
    tgt = e_row[:, 0:1] - oe + 1.0
    one_j = jnp.where((u_rows > 0.5) & (jnp.abs(rank_rows - tgt) < 0.5),
                      1.0, 0.0)
    return one_i + one_j, pm


def _main_body(a_ref, pid_ref, x_ref, cf_ref, s_ref,
               x_out, c_out, s_out, a_out, p_out,
               m_st, u_st, rank_st, meta_st):
    bi = pl.program_id(0)
    bj = pl.program_id(1)

    @pl.when((bi == 0) & (bj == 0))
    def _corner():
        a = a_ref[0]
        row = jax.lax.broadcasted_iota(jnp.int32, (V, V), 0)
        col = jax.lax.broadcasted_iota(jnp.int32, (V, V), 1)
        u = jnp.where((a != 0) & (col > row), 1.0, 0.0).astype(jnp.float32)
        ut = jnp.where(row <= col, 1.0, 0.0).astype(jnp.float32)
        rank = jnp.floor(jnp.dot(u, ut, preferred_element_type=jnp.float32) + 0.5)
        cnt = jnp.sum(u, axis=1)
        sl = jnp.where(row < col, 1.0, 0.0).astype(jnp.float32)
        off = jnp.floor(jnp.sum(cnt[:, None] * sl, axis=0) + 0.5)
        poly = jnp.max(pid_ref[0].astype(jnp.float32), axis=0)
        u_st[...] = u
        rank_st[...] = rank
        meta_st[...] = jnp.concatenate(
            [cnt[None], off[None], poly[None],
             jnp.zeros((5, V), jnp.float32)], axis=0)
        z = jnp.zeros((VB, VB), jnp.float32)
        a_out[0] = z
        p_out[0] = z
        x_out[0] = x_ref[0]
        c_out[0] = cf_ref[0]
        s_out[0] = s_ref[0]

    @pl.when((bi == 0) & (bj > 0))
    def _top_band():
        eb = bj - 1
        m, pm = _build_m(u_st, rank_st, meta_st, eb)
        mt = jnp.transpose(m)
        a_out[0] = mt
        p_out[0] = mt * jnp.transpose(pm)
        m_st[pl.ds(eb, 1)] = m[None]

    @pl.when((bi > 0) & (bj == 0))
    def _left_band():
        eb = bi - 1
        m = m_st[pl.ds(eb, 1)][0]
        _, pm, _, _ = _one_i(meta_st, eb)
        a_out[0] = m
        p_out[0] = m * pm
        x_out[0] = 0.5 * jnp.dot(m, x_ref[0], preferred_element_type=jnp.float32)
        c_out[0] = 0.5 * jnp.dot(m, cf_ref[0], preferred_element_type=jnp.float32)
        s_out[0] = 0.5 * jnp.dot(m, s_ref[0], preferred_element_type=jnp.float32)

    @pl.when((bi > 0) & (bj > 0))
    def _bulk_zero():
        z = jnp.zeros((VB, VB), jnp.float32)
        a_out[0] = z
        p_out[0] = z


def kernel(x_prev, c_prev, A, Pid, s_prev):
    f32 = jnp.float32
    const = lambda i, j: (0, 0, 0)
    rowblk = lambda i, j: (0, i, 0)
    outs = pl.pallas_call(
        _main_body,
        grid=(NB, NB),
        in_specs=[
            pl.BlockSpec((1, V, V), const),
            pl.BlockSpec((1, V, V), const),
            pl.BlockSpec((1, V, FX), const),
            pl.BlockSpec((1, V, FC), const),
            pl.BlockSpec((1, V, FX), const),
        ],
        out_specs=[
            pl.BlockSpec((1, VB, FX), rowblk),
            pl.BlockSpec((1, VB, FC), rowblk),
            pl.BlockSpec((1, VB, FX), rowblk),
            pl.BlockSpec((1, VB, VB), lambda i, j: (0, i, j)),
            pl.BlockSpec((1, VB, VB), lambda i, j: (0, i, j)),
        ],
        out_shape=[
            jax.ShapeDtypeStruct((1, F, FX), f32),
            jax.ShapeDtypeStruct((1, F, FC), f32),
            jax.ShapeDtypeStruct((1, F, FX), f32),
            jax.ShapeDtypeStruct((1, F, F), f32),
            jax.ShapeDtypeStruct((1, F, F), f32),
        ],
        scratch_shapes=[
            pltpu.VMEM((NEB, VB, VB), f32),
            pltpu.VMEM((V, V), f32),
            pltpu.VMEM((V, V), f32),
            pltpu.VMEM((8, V), f32),
        ],
    )(A, Pid, x_prev, c_prev, s_prev)
    x_new, c_new, s_new, a_new, p_new = outs
    return (x_new, c_new, a_new, p_new, s_new)
